# baseline (speedup 1.0000x reference)
import functools

import jax
import jax.numpy as jnp
from jax import lax
from jax.experimental import pallas as pl
from jax.experimental.pallas import tpu as pltpu
from jax.experimental.pallas import tpu_sc as plsc

N_NODES = 10000
N_EDGES = 320000
D_IN = 128
D_EDGE = 16
H = 128
L = 3
OUT = 32

NC = 2
NS = 16
SHARDS = 16
COLS = H // SHARDS
EPT = N_EDGES // NC
K = 640
CHUNKS = EPT // K
GROUPS = K * COLS // 16
AGGR = N_NODES * COLS

BN_SCALE = 0.9999950000374997

ROW_BLK = 2000
NODE_GRID = N_NODES // ROW_BLK
EDGE_BLK = 2000
EDGE_GRID = N_EDGES // EDGE_BLK


def _edge_body(ea_ref, wee_ref, bee_ref, we_ref, be_ref, o_ref):
    we_l = we_ref[0]
    w2 = jnp.dot(wee_ref[...], we_l, preferred_element_type=jnp.float32)
    b2 = jnp.dot(bee_ref[...], we_l, preferred_element_type=jnp.float32)
    b2 = b2 + be_ref[0]
    acc = jnp.dot(ea_ref[...], w2, preferred_element_type=jnp.float32) + b2
    for t in range(SHARDS):
        o_ref[t, 0] = acc[:, t * COLS:(t + 1) * COLS]


def _edge_features(edge_attr, w_ee, b_ee, we, be):
    return pl.pallas_call(
        _edge_body,
        grid=(L, EDGE_GRID),
        in_specs=[
            pl.BlockSpec((EDGE_BLK, D_EDGE), lambda l, i: (i, 0)),
            pl.BlockSpec((D_EDGE, H), lambda l, i: (0, 0)),
            pl.BlockSpec((1, H), lambda l, i: (0, 0)),
            pl.BlockSpec((1, H, H), lambda l, i: (l, 0, 0)),
            pl.BlockSpec((1, 1, H), lambda l, i: (l, 0, 0)),
        ],
        out_specs=pl.BlockSpec((SHARDS, 1, EDGE_BLK, COLS),
                               lambda l, i: (0, l, i, 0)),
        out_shape=jax.ShapeDtypeStruct((SHARDS, L, N_EDGES, COLS), jnp.float32),
    )(edge_attr, w_ee, b_ee, we, be.reshape(L, 1, H))


def _node_update_body(h_ref, p_ref, oneeps_ref, wm1_ref, bm1_ref,
                      wm2_ref, bm2_ref, gamma_ref, beta_ref, o_ref, ht_ref):
    h = h_ref[...]
    a = h * oneeps_ref[...] + p_ref[...]
    t = jnp.dot(a, wm1_ref[...], preferred_element_type=jnp.float32)
    t = jnp.maximum(t + bm1_ref[...], 0.0)
    u = jnp.dot(t, wm2_ref[...], preferred_element_type=jnp.float32)
    u = u + bm2_ref[...]
    u = u * (gamma_ref[...] * BN_SCALE) + beta_ref[...] + h
    hnew = jnp.maximum(u, 0.0)
    o_ref[...] = hnew
    for t2 in range(SHARDS):
        sl = hnew[:, t2 * COLS:(t2 + 1) * COLS]
        ht_ref[t2] = jnp.concatenate([sl, sl], axis=1)


def _node_update(h, p, oneeps, wm1, bm1, wm2, bm2, gamma, beta):
    return pl.pallas_call(
        _node_update_body,
        grid=(NODE_GRID,),
        in_specs=[
            pl.BlockSpec((ROW_BLK, H), lambda i: (i, 0)),
            pl.BlockSpec((ROW_BLK, H), lambda i: (i, 0)),
            pl.BlockSpec((1, H), lambda i: (0, 0)),
            pl.BlockSpec((H, 2 * H), lambda i: (0, 0)),
            pl.BlockSpec((1, 2 * H), lambda i: (0, 0)),
            pl.BlockSpec((2 * H, H), lambda i: (0, 0)),
            pl.BlockSpec((1, H), lambda i: (0, 0)),
            pl.BlockSpec((1, H), lambda i: (0, 0)),
            pl.BlockSpec((1, H), lambda i: (0, 0)),
        ],
        out_specs=[
            pl.BlockSpec((ROW_BLK, H), lambda i: (i, 0)),
            pl.BlockSpec((SHARDS, ROW_BLK, 2 * COLS), lambda i: (0, i, 0)),
        ],
        out_shape=[
            jax.ShapeDtypeStruct((N_NODES, H), jnp.float32),
            jax.ShapeDtypeStruct((SHARDS, N_NODES, 2 * COLS), jnp.float32),
        ],
    )(h, p, oneeps, wm1, bm1, wm2, bm2, gamma, beta)


def _encoder_shard_body(x_ref, w_ref, b_ref, o_ref, ht_ref):
    acc = jnp.dot(x_ref[...], w_ref[...], preferred_element_type=jnp.float32)
    hnew = jnp.maximum(acc + b_ref[...], 0.0)
    o_ref[...] = hnew
    for t in range(SHARDS):
        sl = hnew[:, t * COLS:(t + 1) * COLS]
        ht_ref[t] = jnp.concatenate([sl, sl], axis=1)


def _encode_nodes_sharded(x, w, b):
    return pl.pallas_call(
        _encoder_shard_body,
        grid=(NODE_GRID,),
        in_specs=[
            pl.BlockSpec((ROW_BLK, D_IN), lambda i: (i, 0)),
            pl.BlockSpec((D_IN, H), lambda i: (0, 0)),
            pl.BlockSpec((1, H), lambda i: (0, 0)),
        ],
        out_specs=[
            pl.BlockSpec((ROW_BLK, H), lambda i: (i, 0)),
            pl.BlockSpec((SHARDS, ROW_BLK, 2 * COLS), lambda i: (0, i, 0)),
        ],
        out_shape=[
            jax.ShapeDtypeStruct((N_NODES, H), jnp.float32),
            jax.ShapeDtypeStruct((SHARDS, N_NODES, 2 * COLS), jnp.float32),
        ],
    )(x, w, b)


def _pool_body(h_ref, wo1_ref, bo1_ref, wo2_ref, bo2_ref, logits_ref, pooled_ref):
    xp = jnp.sum(h_ref[...], axis=0, keepdims=True) * (1.0 / N_NODES)
    pooled_ref[...] = xp
    t = jnp.dot(xp, wo1_ref[...], preferred_element_type=jnp.float32)
    t = jnp.maximum(t + bo1_ref[...], 0.0)
    o = jnp.dot(t, wo2_ref[...], preferred_element_type=jnp.float32)
    logits_ref[...] = o + bo2_ref[...]


def _pool_and_project(h, wo1, bo1, wo2, bo2):
    return pl.pallas_call(
        _pool_body,
        out_shape=[
            jax.ShapeDtypeStruct((1, OUT), jnp.float32),
            jax.ShapeDtypeStruct((1, H), jnp.float32),
        ],
    )(h, wo1, bo1, wo2, bo2)


@functools.lru_cache(maxsize=None)
def _make_msgpass(layer):
    mesh = plsc.VectorSubcoreMesh(core_axis_name="c", subcore_axis_name="s",
                                  num_cores=NC, num_subcores=NS)

    @functools.partial(
        pl.kernel,
        out_type=jax.ShapeDtypeStruct((NC * SHARDS * AGGR,), jnp.float32),
        mesh=mesh,
        compiler_params=pltpu.CompilerParams(use_tc_tiling_on_sc=False, needs_layout_passes=False),
        scratch_types=[
            pltpu.VMEM((K,), jnp.int32),
            pltpu.VMEM((K, 2 * COLS), jnp.float32),
            pltpu.VMEM((K * COLS,), jnp.float32),
            pltpu.VMEM((GROUPS, 16), jnp.int32),
            pltpu.VMEM((K * COLS,), jnp.float32),
            pltpu.VMEM((AGGR,), jnp.float32),
        ],
    )
    def msgpass(ht_hbm, et_hbm, src_hbm, fidx_hbm, out_hbm,
                sidx, hbuf, ebuf, fbuf, mbuf, aggr):
        c = lax.axis_index("c")
        s = lax.axis_index("s")

        @plsc.parallel_loop(0, AGGR // 16, unroll=8)
        def _zero(i):
            aggr[pl.ds(i * 16, 16)] = jnp.zeros((16,), jnp.float32)

        lowmask = lax.iota(jnp.int32, 16) < 8
        shard_off = s * N_NODES
        ebase = ((s * L + layer) * N_EDGES + c * EPT) * COLS
        sbase = c * EPT
        fbase = c * (EPT // 2)

        def chunk(ci, carry):
            off = ci * K
            soff = pl.multiple_of(sbase + off, 8)
            eoff = pl.multiple_of(ebase + off * COLS, 8)
            foff = pl.multiple_of(fbase + off // 2, 8)
            pltpu.sync_copy(src_hbm.at[pl.ds(soff, K)], sidx)
            pltpu.sync_copy(et_hbm.at[pl.ds(eoff, K * COLS)], ebuf)
            pltpu.sync_copy(fidx_hbm.at[pl.ds(foff, GROUPS)], fbuf)

            @plsc.parallel_loop(0, K // 16, unroll=4)
            def _adjust(g):
                sidx[pl.ds(g * 16, 16)] = sidx[pl.ds(g * 16, 16)] + shard_off

            pltpu.sync_copy(ht_hbm.at[sidx], hbuf)

            @plsc.parallel_loop(0, GROUPS, unroll=8)
            def _msg(g):
                vh = jnp.where(lowmask, hbuf[2 * g, :], hbuf[2 * g + 1, :])
                ve = ebuf[pl.ds(g * 16, 16)]
                mbuf[pl.ds(g * 16, 16)] = jnp.maximum(vh + ve, 0.0)

            @plsc.parallel_loop(0, GROUPS, unroll=8)
            def _accum(g):
                plsc.addupdate_scatter(aggr, [fbuf[g, :]], mbuf[pl.ds(g * 16, 16)])

            return carry

        lax.fori_loop(0, CHUNKS, chunk, 0)
        wid = c * NS + s
        pltpu.sync_copy(aggr, out_hbm.at[pl.ds(wid * AGGR, AGGR)])

    return msgpass


def _aggregate(h_sharded, e_all, src, fidx, layer):
    parts = _make_msgpass(layer)(h_sharded, e_all, src, fidx)
    parts = parts.reshape(NC, SHARDS, N_NODES, COLS)
    agg = parts[0] + parts[1]
    return agg.transpose(1, 0, 2).reshape(N_NODES, H)


def kernel(x, edge_index, edge_attr, W_ne, b_ne, W_ee, b_ee, eps, We, be,
           Wm1, bm1, Wm2, bm2, gamma, beta, Wo1, bo1, Wo2, bo2):
    src = edge_index[0].astype(jnp.int32)
    dst = edge_index[1].astype(jnp.int32)
    fidx = (dst.reshape(-1, 2, 1) * COLS
            + jnp.arange(COLS, dtype=jnp.int32)).reshape(-1, 16)

    b_ne2 = b_ne.reshape(1, H)
    b_ee2 = b_ee.reshape(1, H)
    bo1_2 = bo1.reshape(1, H // 2)
    bo2_2 = bo2.reshape(1, OUT)

    h, h_sh = _encode_nodes_sharded(x, W_ne, b_ne2)
    e_all = _edge_features(edge_attr, W_ee, b_ee2, We, be)
    e_all = e_all.reshape(SHARDS * L * N_EDGES * COLS)

    for l in range(L):
        aggr = _aggregate(h_sh.reshape(SHARDS * N_NODES, 2 * COLS), e_all, src,
                          fidx, l)
        oneeps = jnp.broadcast_to((1.0 + eps[l]).reshape(1, 1), (1, H))
        h, h_sh = _node_update(h, aggr, oneeps,
                               Wm1[l], bm1[l].reshape(1, 2 * H),
                               Wm2[l], bm2[l].reshape(1, H),
                               gamma[l].reshape(1, H), beta[l].reshape(1, H))

    out_logits, x_pooled = _pool_and_project(h, Wo1, bo1_2, Wo2, bo2_2)
    return out_logits, x_pooled


# speedup vs baseline: 1.0921x; 1.0921x over previous
import functools

import jax
import jax.numpy as jnp
from jax import lax
from jax.experimental import pallas as pl
from jax.experimental.pallas import tpu as pltpu
from jax.experimental.pallas import tpu_sc as plsc

N_NODES = 10000
N_EDGES = 320000
D_IN = 128
D_EDGE = 16
H = 128
L = 3
OUT = 32

NC = 2
NS = 16
SHARDS = 16
COLS = H // SHARDS
EPT = N_EDGES // NC
K = 640
CHUNKS = EPT // K
GROUPS = K * COLS // 16
AGGR = N_NODES * COLS

BN_SCALE = 0.9999950000374997

ROW_BLK = 2000
NODE_GRID = N_NODES // ROW_BLK
EDGE_BLK = 2000
EDGE_GRID = N_EDGES // EDGE_BLK


def _edge_body(ea_ref, wee_ref, bee_ref, we_ref, be_ref, o_ref):
    we_l = we_ref[0]
    w2 = jnp.dot(wee_ref[...], we_l, preferred_element_type=jnp.float32)
    b2 = jnp.dot(bee_ref[...], we_l, preferred_element_type=jnp.float32)
    b2 = b2 + be_ref[0]
    acc = jnp.dot(ea_ref[...], w2, preferred_element_type=jnp.float32) + b2
    for t in range(SHARDS):
        o_ref[t, 0] = acc[:, t * COLS:(t + 1) * COLS]


def _edge_features(edge_attr, w_ee, b_ee, we, be):
    return pl.pallas_call(
        _edge_body,
        grid=(L, EDGE_GRID),
        in_specs=[
            pl.BlockSpec((EDGE_BLK, D_EDGE), lambda l, i: (i, 0)),
            pl.BlockSpec((D_EDGE, H), lambda l, i: (0, 0)),
            pl.BlockSpec((1, H), lambda l, i: (0, 0)),
            pl.BlockSpec((1, H, H), lambda l, i: (l, 0, 0)),
            pl.BlockSpec((1, 1, H), lambda l, i: (l, 0, 0)),
        ],
        out_specs=pl.BlockSpec((SHARDS, 1, EDGE_BLK, COLS),
                               lambda l, i: (0, l, i, 0)),
        out_shape=jax.ShapeDtypeStruct((SHARDS, L, N_EDGES, COLS), jnp.float32),
    )(edge_attr, w_ee, b_ee, we, be.reshape(L, 1, H))


def _node_update_body(h_ref, p_ref, oneeps_ref, wm1_ref, bm1_ref,
                      wm2_ref, bm2_ref, gamma_ref, beta_ref, o_ref, ht_ref):
    h = h_ref[...]
    a = h * oneeps_ref[...] + p_ref[...]
    t = jnp.dot(a, wm1_ref[...], preferred_element_type=jnp.float32)
    t = jnp.maximum(t + bm1_ref[...], 0.0)
    u = jnp.dot(t, wm2_ref[...], preferred_element_type=jnp.float32)
    u = u + bm2_ref[...]
    u = u * (gamma_ref[...] * BN_SCALE) + beta_ref[...] + h
    hnew = jnp.maximum(u, 0.0)
    o_ref[...] = hnew
    for t2 in range(SHARDS):
        sl = hnew[:, t2 * COLS:(t2 + 1) * COLS]
        ht_ref[t2] = jnp.concatenate([sl, sl], axis=1)


def _node_update(h, p, oneeps, wm1, bm1, wm2, bm2, gamma, beta):
    return pl.pallas_call(
        _node_update_body,
        grid=(NODE_GRID,),
        in_specs=[
            pl.BlockSpec((ROW_BLK, H), lambda i: (i, 0)),
            pl.BlockSpec((ROW_BLK, H), lambda i: (i, 0)),
            pl.BlockSpec((1, H), lambda i: (0, 0)),
            pl.BlockSpec((H, 2 * H), lambda i: (0, 0)),
            pl.BlockSpec((1, 2 * H), lambda i: (0, 0)),
            pl.BlockSpec((2 * H, H), lambda i: (0, 0)),
            pl.BlockSpec((1, H), lambda i: (0, 0)),
            pl.BlockSpec((1, H), lambda i: (0, 0)),
            pl.BlockSpec((1, H), lambda i: (0, 0)),
        ],
        out_specs=[
            pl.BlockSpec((ROW_BLK, H), lambda i: (i, 0)),
            pl.BlockSpec((SHARDS, ROW_BLK, 2 * COLS), lambda i: (0, i, 0)),
        ],
        out_shape=[
            jax.ShapeDtypeStruct((N_NODES, H), jnp.float32),
            jax.ShapeDtypeStruct((SHARDS, N_NODES, 2 * COLS), jnp.float32),
        ],
    )(h, p, oneeps, wm1, bm1, wm2, bm2, gamma, beta)


def _encoder_shard_body(x_ref, w_ref, b_ref, o_ref, ht_ref):
    acc = jnp.dot(x_ref[...], w_ref[...], preferred_element_type=jnp.float32)
    hnew = jnp.maximum(acc + b_ref[...], 0.0)
    o_ref[...] = hnew
    for t in range(SHARDS):
        sl = hnew[:, t * COLS:(t + 1) * COLS]
        ht_ref[t] = jnp.concatenate([sl, sl], axis=1)


def _encode_nodes_sharded(x, w, b):
    return pl.pallas_call(
        _encoder_shard_body,
        grid=(NODE_GRID,),
        in_specs=[
            pl.BlockSpec((ROW_BLK, D_IN), lambda i: (i, 0)),
            pl.BlockSpec((D_IN, H), lambda i: (0, 0)),
            pl.BlockSpec((1, H), lambda i: (0, 0)),
        ],
        out_specs=[
            pl.BlockSpec((ROW_BLK, H), lambda i: (i, 0)),
            pl.BlockSpec((SHARDS, ROW_BLK, 2 * COLS), lambda i: (0, i, 0)),
        ],
        out_shape=[
            jax.ShapeDtypeStruct((N_NODES, H), jnp.float32),
            jax.ShapeDtypeStruct((SHARDS, N_NODES, 2 * COLS), jnp.float32),
        ],
    )(x, w, b)


def _pool_body(h_ref, wo1_ref, bo1_ref, wo2_ref, bo2_ref, logits_ref, pooled_ref):
    xp = jnp.sum(h_ref[...], axis=0, keepdims=True) * (1.0 / N_NODES)
    pooled_ref[...] = xp
    t = jnp.dot(xp, wo1_ref[...], preferred_element_type=jnp.float32)
    t = jnp.maximum(t + bo1_ref[...], 0.0)
    o = jnp.dot(t, wo2_ref[...], preferred_element_type=jnp.float32)
    logits_ref[...] = o + bo2_ref[...]


def _pool_and_project(h, wo1, bo1, wo2, bo2):
    return pl.pallas_call(
        _pool_body,
        out_shape=[
            jax.ShapeDtypeStruct((1, OUT), jnp.float32),
            jax.ShapeDtypeStruct((1, H), jnp.float32),
        ],
    )(h, wo1, bo1, wo2, bo2)


@functools.lru_cache(maxsize=None)
def _make_msgpass(layer):
    mesh = plsc.VectorSubcoreMesh(core_axis_name="c", subcore_axis_name="s",
                                  num_cores=NC, num_subcores=NS)

    @functools.partial(
        pl.kernel,
        out_type=jax.ShapeDtypeStruct((NC * SHARDS * AGGR,), jnp.float32),
        mesh=mesh,
        compiler_params=pltpu.CompilerParams(use_tc_tiling_on_sc=False, needs_layout_passes=False),
        scratch_types=[
            pltpu.VMEM((K,), jnp.int32),
            pltpu.VMEM((K, 2 * COLS), jnp.float32),
            pltpu.VMEM((K * COLS,), jnp.float32),
            pltpu.VMEM((GROUPS, 16), jnp.int32),
            pltpu.VMEM((K * COLS,), jnp.float32),
            pltpu.VMEM((AGGR,), jnp.float32),
            pltpu.SemaphoreType.DMA,
            pltpu.SemaphoreType.DMA,
        ],
    )
    def msgpass(ht_hbm, et_hbm, src_hbm, fidx_hbm, out_hbm,
                sidx, hbuf, ebuf, fbuf, mbuf, aggr, esem, fsem):
        c = lax.axis_index("c")
        s = lax.axis_index("s")

        @plsc.parallel_loop(0, AGGR // 16, unroll=8)
        def _zero(i):
            aggr[pl.ds(i * 16, 16)] = jnp.zeros((16,), jnp.float32)

        lowmask = lax.iota(jnp.int32, 16) < 8
        shard_off = s * N_NODES
        ebase = ((s * L + layer) * N_EDGES + c * EPT) * COLS
        sbase = c * EPT
        fbase = c * (EPT // 2)

        def chunk(ci, carry):
            off = ci * K
            soff = pl.multiple_of(sbase + off, 8)
            eoff = pl.multiple_of(ebase + off * COLS, 8)
            foff = pl.multiple_of(fbase + off // 2, 8)
            ecopy = pltpu.async_copy(et_hbm.at[pl.ds(eoff, K * COLS)], ebuf,
                                     esem)
            fcopy = pltpu.async_copy(fidx_hbm.at[pl.ds(foff, GROUPS)], fbuf,
                                     fsem)
            pltpu.sync_copy(src_hbm.at[pl.ds(soff, K)], sidx)

            @plsc.parallel_loop(0, K // 16, unroll=4)
            def _adjust(g):
                sidx[pl.ds(g * 16, 16)] = sidx[pl.ds(g * 16, 16)] + shard_off

            pltpu.sync_copy(ht_hbm.at[sidx], hbuf)
            ecopy.wait()
            fcopy.wait()

            @plsc.parallel_loop(0, GROUPS, unroll=8)
            def _msg(g):
                vh = jnp.where(lowmask, hbuf[2 * g, :], hbuf[2 * g + 1, :])
                ve = ebuf[pl.ds(g * 16, 16)]
                mbuf[pl.ds(g * 16, 16)] = jnp.maximum(vh + ve, 0.0)

            @plsc.parallel_loop(0, GROUPS, unroll=8)
            def _accum(g):
                plsc.addupdate_scatter(aggr, [fbuf[g, :]], mbuf[pl.ds(g * 16, 16)])

            return carry

        lax.fori_loop(0, CHUNKS, chunk, 0)
        wid = c * NS + s
        pltpu.sync_copy(aggr, out_hbm.at[pl.ds(wid * AGGR, AGGR)])

    return msgpass


def _aggregate(h_sharded, e_all, src, fidx, layer):
    parts = _make_msgpass(layer)(h_sharded, e_all, src, fidx)
    parts = parts.reshape(NC, SHARDS, N_NODES, COLS)
    agg = parts[0] + parts[1]
    return agg.transpose(1, 0, 2).reshape(N_NODES, H)


def kernel(x, edge_index, edge_attr, W_ne, b_ne, W_ee, b_ee, eps, We, be,
           Wm1, bm1, Wm2, bm2, gamma, beta, Wo1, bo1, Wo2, bo2):
    src = edge_index[0].astype(jnp.int32)
    dst = edge_index[1].astype(jnp.int32)
    fidx = (dst.reshape(-1, 2, 1) * COLS
            + jnp.arange(COLS, dtype=jnp.int32)).reshape(-1, 16)

    b_ne2 = b_ne.reshape(1, H)
    b_ee2 = b_ee.reshape(1, H)
    bo1_2 = bo1.reshape(1, H // 2)
    bo2_2 = bo2.reshape(1, OUT)

    h, h_sh = _encode_nodes_sharded(x, W_ne, b_ne2)
    e_all = _edge_features(edge_attr, W_ee, b_ee2, We, be)
    e_all = e_all.reshape(SHARDS * L * N_EDGES * COLS)

    for l in range(L):
        aggr = _aggregate(h_sh.reshape(SHARDS * N_NODES, 2 * COLS), e_all, src,
                          fidx, l)
        oneeps = jnp.broadcast_to((1.0 + eps[l]).reshape(1, 1), (1, H))
        h, h_sh = _node_update(h, aggr, oneeps,
                               Wm1[l], bm1[l].reshape(1, 2 * H),
                               Wm2[l], bm2[l].reshape(1, H),
                               gamma[l].reshape(1, H), beta[l].reshape(1, H))

    out_logits, x_pooled = _pool_and_project(h, Wo1, bo1_2, Wo2, bo2_2)
    return out_logits, x_pooled


# speedup vs baseline: 1.1707x; 1.0720x over previous
import functools

import jax
import jax.numpy as jnp
from jax import lax
from jax.experimental import pallas as pl
from jax.experimental.pallas import tpu as pltpu
from jax.experimental.pallas import tpu_sc as plsc

N_NODES = 10000
N_EDGES = 320000
D_IN = 128
D_EDGE = 16
H = 128
L = 3
OUT = 32

NC = 2
NS = 16
SHARDS = 16
COLS = H // SHARDS
EPT = N_EDGES // NC
K = 320
CHUNKS = EPT // K
GROUPS = K * COLS // 16
AGGR = N_NODES * COLS

BN_SCALE = 0.9999950000374997

ROW_BLK = 2000
NODE_GRID = N_NODES // ROW_BLK
EDGE_BLK = 2000
EDGE_GRID = N_EDGES // EDGE_BLK


def _edge_body(ea_ref, wee_ref, bee_ref, we_ref, be_ref, o_ref):
    we_l = we_ref[0]
    w2 = jnp.dot(wee_ref[...], we_l, preferred_element_type=jnp.float32)
    b2 = jnp.dot(bee_ref[...], we_l, preferred_element_type=jnp.float32)
    b2 = b2 + be_ref[0]
    acc = jnp.dot(ea_ref[...], w2, preferred_element_type=jnp.float32) + b2
    for t in range(SHARDS):
        o_ref[t, 0] = acc[:, t * COLS:(t + 1) * COLS]


def _edge_features(edge_attr, w_ee, b_ee, we, be):
    return pl.pallas_call(
        _edge_body,
        grid=(L, EDGE_GRID),
        in_specs=[
            pl.BlockSpec((EDGE_BLK, D_EDGE), lambda l, i: (i, 0)),
            pl.BlockSpec((D_EDGE, H), lambda l, i: (0, 0)),
            pl.BlockSpec((1, H), lambda l, i: (0, 0)),
            pl.BlockSpec((1, H, H), lambda l, i: (l, 0, 0)),
            pl.BlockSpec((1, 1, H), lambda l, i: (l, 0, 0)),
        ],
        out_specs=pl.BlockSpec((SHARDS, 1, EDGE_BLK, COLS),
                               lambda l, i: (0, l, i, 0)),
        out_shape=jax.ShapeDtypeStruct((SHARDS, L, N_EDGES, COLS), jnp.float32),
    )(edge_attr, w_ee, b_ee, we, be.reshape(L, 1, H))


def _node_update_body(h_ref, p_ref, oneeps_ref, wm1_ref, bm1_ref,
                      wm2_ref, bm2_ref, gamma_ref, beta_ref, o_ref, ht_ref):
    h = h_ref[...]
    a = h * oneeps_ref[...] + p_ref[...]
    t = jnp.dot(a, wm1_ref[...], preferred_element_type=jnp.float32)
    t = jnp.maximum(t + bm1_ref[...], 0.0)
    u = jnp.dot(t, wm2_ref[...], preferred_element_type=jnp.float32)
    u = u + bm2_ref[...]
    u = u * (gamma_ref[...] * BN_SCALE) + beta_ref[...] + h
    hnew = jnp.maximum(u, 0.0)
    o_ref[...] = hnew
    for t2 in range(SHARDS):
        sl = hnew[:, t2 * COLS:(t2 + 1) * COLS]
        ht_ref[t2] = jnp.concatenate([sl, sl], axis=1)


def _node_update(h, p, oneeps, wm1, bm1, wm2, bm2, gamma, beta):
    return pl.pallas_call(
        _node_update_body,
        grid=(NODE_GRID,),
        in_specs=[
            pl.BlockSpec((ROW_BLK, H), lambda i: (i, 0)),
            pl.BlockSpec((ROW_BLK, H), lambda i: (i, 0)),
            pl.BlockSpec((1, H), lambda i: (0, 0)),
            pl.BlockSpec((H, 2 * H), lambda i: (0, 0)),
            pl.BlockSpec((1, 2 * H), lambda i: (0, 0)),
            pl.BlockSpec((2 * H, H), lambda i: (0, 0)),
            pl.BlockSpec((1, H), lambda i: (0, 0)),
            pl.BlockSpec((1, H), lambda i: (0, 0)),
            pl.BlockSpec((1, H), lambda i: (0, 0)),
        ],
        out_specs=[
            pl.BlockSpec((ROW_BLK, H), lambda i: (i, 0)),
            pl.BlockSpec((SHARDS, ROW_BLK, 2 * COLS), lambda i: (0, i, 0)),
        ],
        out_shape=[
            jax.ShapeDtypeStruct((N_NODES, H), jnp.float32),
            jax.ShapeDtypeStruct((SHARDS, N_NODES, 2 * COLS), jnp.float32),
        ],
    )(h, p, oneeps, wm1, bm1, wm2, bm2, gamma, beta)


def _encoder_shard_body(x_ref, w_ref, b_ref, o_ref, ht_ref):
    acc = jnp.dot(x_ref[...], w_ref[...], preferred_element_type=jnp.float32)
    hnew = jnp.maximum(acc + b_ref[...], 0.0)
    o_ref[...] = hnew
    for t in range(SHARDS):
        sl = hnew[:, t * COLS:(t + 1) * COLS]
        ht_ref[t] = jnp.concatenate([sl, sl], axis=1)


def _encode_nodes_sharded(x, w, b):
    return pl.pallas_call(
        _encoder_shard_body,
        grid=(NODE_GRID,),
        in_specs=[
            pl.BlockSpec((ROW_BLK, D_IN), lambda i: (i, 0)),
            pl.BlockSpec((D_IN, H), lambda i: (0, 0)),
            pl.BlockSpec((1, H), lambda i: (0, 0)),
        ],
        out_specs=[
            pl.BlockSpec((ROW_BLK, H), lambda i: (i, 0)),
            pl.BlockSpec((SHARDS, ROW_BLK, 2 * COLS), lambda i: (0, i, 0)),
        ],
        out_shape=[
            jax.ShapeDtypeStruct((N_NODES, H), jnp.float32),
            jax.ShapeDtypeStruct((SHARDS, N_NODES, 2 * COLS), jnp.float32),
        ],
    )(x, w, b)


def _pool_body(h_ref, wo1_ref, bo1_ref, wo2_ref, bo2_ref, logits_ref, pooled_ref):
    xp = jnp.sum(h_ref[...], axis=0, keepdims=True) * (1.0 / N_NODES)
    pooled_ref[...] = xp
    t = jnp.dot(xp, wo1_ref[...], preferred_element_type=jnp.float32)
    t = jnp.maximum(t + bo1_ref[...], 0.0)
    o = jnp.dot(t, wo2_ref[...], preferred_element_type=jnp.float32)
    logits_ref[...] = o + bo2_ref[...]


def _pool_and_project(h, wo1, bo1, wo2, bo2):
    return pl.pallas_call(
        _pool_body,
        out_shape=[
            jax.ShapeDtypeStruct((1, OUT), jnp.float32),
            jax.ShapeDtypeStruct((1, H), jnp.float32),
        ],
    )(h, wo1, bo1, wo2, bo2)


@functools.lru_cache(maxsize=None)
def _make_msgpass(layer):
    mesh = plsc.VectorSubcoreMesh(core_axis_name="c", subcore_axis_name="s",
                                  num_cores=NC, num_subcores=NS)

    @functools.partial(
        pl.kernel,
        out_type=jax.ShapeDtypeStruct((NC * SHARDS * AGGR,), jnp.float32),
        mesh=mesh,
        compiler_params=pltpu.CompilerParams(use_tc_tiling_on_sc=False,
                                             needs_layout_passes=False),
        scratch_types=[
            pltpu.VMEM((K,), jnp.int32),
            pltpu.VMEM((K,), jnp.int32),
            pltpu.VMEM((K, 2 * COLS), jnp.float32),
            pltpu.VMEM((K, 2 * COLS), jnp.float32),
            pltpu.VMEM((K * COLS,), jnp.float32),
            pltpu.VMEM((K * COLS,), jnp.float32),
            pltpu.VMEM((GROUPS, 16), jnp.int32),
            pltpu.VMEM((GROUPS, 16), jnp.int32),
            pltpu.VMEM((AGGR,), jnp.float32),
            pltpu.SemaphoreType.DMA,
            pltpu.SemaphoreType.DMA,
            pltpu.SemaphoreType.DMA,
            pltpu.SemaphoreType.DMA,
            pltpu.SemaphoreType.DMA,
            pltpu.SemaphoreType.DMA,
        ],
    )
    def msgpass(ht_hbm, et_hbm, src_hbm, fidx_hbm, out_hbm,
                sidx0, sidx1, hbuf0, hbuf1, ebuf0, ebuf1, fbuf0, fbuf1,
                aggr, es0, es1, fs0, fs1, gs0, gs1):
        c = lax.axis_index("c")
        s = lax.axis_index("s")

        @plsc.parallel_loop(0, AGGR // 16, unroll=8)
        def _zero(i):
            aggr[pl.ds(i * 16, 16)] = jnp.zeros((16,), jnp.float32)

        lowmask = lax.iota(jnp.int32, 16) < 8
        shard_off = s * N_NODES
        ebase = ((s * L + layer) * N_EDGES + c * EPT) * COLS
        sbase = c * EPT
        fbase = c * (EPT // 2)

        bufs = ((sidx0, hbuf0, ebuf0, fbuf0, es0, fs0, gs0),
                (sidx1, hbuf1, ebuf1, fbuf1, es1, fs1, gs1))

        def fire(ci, b):
            sidx, hbuf, ebuf, fbuf, esem, fsem, gsem = bufs[b]
            off = ci * K
            soff = pl.multiple_of(sbase + off, 8)
            eoff = pl.multiple_of(ebase + off * COLS, 8)
            foff = pl.multiple_of(fbase + off // 2, 8)
            pltpu.async_copy(et_hbm.at[pl.ds(eoff, K * COLS)], ebuf, esem)
            pltpu.async_copy(fidx_hbm.at[pl.ds(foff, GROUPS)], fbuf, fsem)
            pltpu.sync_copy(src_hbm.at[pl.ds(soff, K)], sidx)

            @plsc.parallel_loop(0, K // 16, unroll=4)
            def _adjust(g):
                sidx[pl.ds(g * 16, 16)] = sidx[pl.ds(g * 16, 16)] + shard_off

            pltpu.async_copy(ht_hbm.at[sidx], hbuf, gsem)

        def drain_compute(b):
            sidx, hbuf, ebuf, fbuf, esem, fsem, gsem = bufs[b]
            pltpu.make_async_copy(ht_hbm.at[sidx], hbuf, gsem).wait()
            pltpu.make_async_copy(et_hbm.at[pl.ds(0, K * COLS)], ebuf,
                                  esem).wait()
            pltpu.make_async_copy(fidx_hbm.at[pl.ds(0, GROUPS)], fbuf,
                                  fsem).wait()

            @plsc.parallel_loop(0, GROUPS, unroll=8)
            def _accum(g):
                vh = jnp.where(lowmask, hbuf[2 * g, :], hbuf[2 * g + 1, :])
                ve = ebuf[pl.ds(g * 16, 16)]
                v = jnp.maximum(vh + ve, 0.0)
                plsc.addupdate_scatter(aggr, [fbuf[g, :]], v)

        fire(0, 0)

        def body(i, carry):
            fire(2 * i + 1, 1)
            drain_compute(0)
            fire(2 * i + 2, 0)
            drain_compute(1)
            return carry

        lax.fori_loop(0, CHUNKS // 2 - 1, body, 0)
        fire(CHUNKS - 1, 1)
        drain_compute(0)
        drain_compute(1)

        wid = c * NS + s
        pltpu.sync_copy(aggr, out_hbm.at[pl.ds(wid * AGGR, AGGR)])

    return msgpass


def _aggregate(h_sharded, e_all, src, fidx, layer):
    parts = _make_msgpass(layer)(h_sharded, e_all, src, fidx)
    parts = parts.reshape(NC, SHARDS, N_NODES, COLS)
    agg = parts[0] + parts[1]
    return agg.transpose(1, 0, 2).reshape(N_NODES, H)


def kernel(x, edge_index, edge_attr, W_ne, b_ne, W_ee, b_ee, eps, We, be,
           Wm1, bm1, Wm2, bm2, gamma, beta, Wo1, bo1, Wo2, bo2):
    src = edge_index[0].astype(jnp.int32)
    dst = edge_index[1].astype(jnp.int32)
    fidx = (dst.reshape(-1, 2, 1) * COLS
            + jnp.arange(COLS, dtype=jnp.int32)).reshape(-1, 16)

    b_ne2 = b_ne.reshape(1, H)
    b_ee2 = b_ee.reshape(1, H)
    bo1_2 = bo1.reshape(1, H // 2)
    bo2_2 = bo2.reshape(1, OUT)

    h, h_sh = _encode_nodes_sharded(x, W_ne, b_ne2)
    e_all = _edge_features(edge_attr, W_ee, b_ee2, We, be)
    e_all = e_all.reshape(SHARDS * L * N_EDGES * COLS)

    for l in range(L):
        aggr = _aggregate(h_sh.reshape(SHARDS * N_NODES, 2 * COLS), e_all, src,
                          fidx, l)
        oneeps = jnp.broadcast_to((1.0 + eps[l]).reshape(1, 1), (1, H))
        h, h_sh = _node_update(h, aggr, oneeps,
                               Wm1[l], bm1[l].reshape(1, 2 * H),
                               Wm2[l], bm2[l].reshape(1, H),
                               gamma[l].reshape(1, H), beta[l].reshape(1, H))

    out_logits, x_pooled = _pool_and_project(h, Wo1, bo1_2, Wo2, bo2_2)
    return out_logits, x_pooled
